# baseline (speedup 1.0000x reference)
import functools

import jax
import jax.numpy as jnp
from jax import lax
from jax.experimental import pallas as pl
from jax.experimental.pallas import tpu as pltpu
from jax.experimental.pallas import tpu_sc as plsc

B, H, W, C = 4, 224, 224, 96
OUT_H, OUT_W = 224, 224
NPIX = B * OUT_H * OUT_W
NW = 32
PIX_PER_TILE = NPIX // NW
CH = 64
N_CHUNKS = PIX_PER_TILE // CH
TROW = 128
N_TROWS = NPIX // TROW
XP = 128
ROWS_PER_STEP = 32
CV = C // 16


def _pool_body(x_ref, out_ref, xp_ref):
    b = pl.program_id(0)
    h = pl.program_id(1)
    xb = x_ref[...]
    s = jnp.sum(xb, axis=(0, 1, 2)) * (1.0 / (H * W))
    sel = lax.broadcasted_iota(jnp.int32, (B, C), 0) == b
    contrib = jnp.where(sel, s.reshape(1, C), 0.0)

    @pl.when((b == 0) & (h == 0))
    def _():
        out_ref[...] = contrib

    @pl.when((b != 0) | (h != 0))
    def _():
        out_ref[...] = out_ref[...] + contrib

    xp_ref[:, 0:C] = xb.reshape(xp_ref.shape[0], C)


def _pooled(X):
    hc = 28
    steps = H // hc
    rows = hc * W
    return pl.pallas_call(
        _pool_body,
        grid=(B, steps),
        in_specs=[pl.BlockSpec((1, hc, W, C), lambda b, h: (b, h, 0, 0))],
        out_specs=[
            pl.BlockSpec((B, C), lambda b, h: (0, 0)),
            pl.BlockSpec((rows, XP), lambda b, h: (b * steps + h, 0)),
        ],
        out_shape=[
            jax.ShapeDtypeStruct((B, C), jnp.float32),
            jax.ShapeDtypeStruct((NPIX, XP), jnp.float32),
        ],
    )(X)


def _grid_body(pooled_ref, wloc_ref, bloc_ref, idx_ref, wt_ref):
    b = pl.program_id(0)
    def _bf(v):
        return v.astype(jnp.bfloat16).astype(jnp.float32)

    th_all = jnp.dot(pooled_ref[...].astype(jnp.bfloat16),
                     wloc_ref[...].astype(jnp.bfloat16),
                     preferred_element_type=jnp.float32) + bloc_ref[...]
    sel = lax.broadcasted_iota(jnp.int32, (B, 6), 0) == b
    th = jnp.sum(jnp.where(sel, th_all, 0.0), axis=0)
    t00, t01, t02 = _bf(th[0]), _bf(th[1]), _bf(th[2])
    t10, t11, t12 = _bf(th[3]), _bf(th[4]), _bf(th[5])

    shp = (NPIX // B // TROW, TROW)
    p_loc = (lax.broadcasted_iota(jnp.int32, shp, 0) * TROW
             + lax.broadcasted_iota(jnp.int32, shp, 1))
    p2 = p_loc >> 5
    row_i = (p2 * 9363) >> 16
    col_i = p_loc - row_i * OUT_W
    col = col_i.astype(jnp.float32)
    row = row_i.astype(jnp.float32)
    sc = col / jnp.float32(OUT_W - 1)
    sr = row / jnp.float32(OUT_H - 1)
    xg = _bf(sc - (1.0 - sc))
    yg = _bf(sr - (1.0 - sr))
    x_s = t00 * xg + t01 * yg + t02
    y_s = t10 * xg + t11 * yg + t12
    x = 0.5 * (x_s + 1.0) * jnp.float32(W)
    y = 0.5 * (y_s + 1.0) * jnp.float32(H)

    xf = jnp.floor(x)
    yf = jnp.floor(y)
    x0f = jnp.clip(xf, 0.0, W - 1)
    x1f = jnp.clip(xf + 1.0, 0.0, W - 1)
    y0f = jnp.clip(yf, 0.0, H - 1)
    y1f = jnp.clip(yf + 1.0, 0.0, H - 1)
    x0 = x0f.astype(jnp.int32)
    x1 = x1f.astype(jnp.int32)
    y0 = y0f.astype(jnp.int32)
    y1 = y1f.astype(jnp.int32)

    wa = (x1f - x) * (y1f - y)
    wb = (x1f - x) * (y - y0f)
    wc = (x - x0f) * (y1f - y)
    wd = (x - x0f) * (y - y0f)

    base = b * (H * W)
    ia = base + y0 * W + x0
    ib = base + y1 * W + x0
    ic = base + y0 * W + x1
    id_ = base + y1 * W + x1

    idx_ref[...] = jnp.stack([ia, ib, ic, id_], axis=0)
    wt_ref[...] = jnp.stack([wa, wb, wc, wd], axis=0)


def _grid_tables(pooled, W_loc, b_loc2):
    cpb = NPIX // B // TROW
    return pl.pallas_call(
        _grid_body,
        grid=(B,),
        in_specs=[
            pl.BlockSpec((B, C), lambda b: (0, 0)),
            pl.BlockSpec((C, 6), lambda b: (0, 0)),
            pl.BlockSpec((1, 6), lambda b: (0, 0)),
        ],
        out_specs=[
            pl.BlockSpec((4, cpb, TROW), lambda b: (0, b, 0)),
            pl.BlockSpec((4, cpb, TROW), lambda b: (0, b, 0)),
        ],
        out_shape=[
            jax.ShapeDtypeStruct((4, N_TROWS, TROW), jnp.int32),
            jax.ShapeDtypeStruct((4, N_TROWS, TROW), jnp.float32),
        ],
    )(pooled, W_loc, b_loc2)


def _sc_body(xflat, idx_hbm, wt_hbm, out_hbm, *s):
    slots = (s[:7], s[7:14])
    ob, osem = s[14], s[15]
    wid = lax.axis_index("s") * 2 + lax.axis_index("c")
    tile_base = wid * PIX_PER_TILE

    def load_and_fire(ci, slot):
        iv, wv, g0, g1, g2, g3, sem = slot
        g = wid * N_CHUNKS + ci
        gr = g // 2
        off = (g % 2) * CH
        pltpu.sync_copy(idx_hbm.at[:, gr, pl.ds(off, CH)], iv)
        pltpu.sync_copy(wt_hbm.at[:, gr, pl.ds(off, CH)], wv)
        pltpu.async_copy(xflat.at[iv.at[0]], g0, sem)
        pltpu.async_copy(xflat.at[iv.at[1]], g1, sem)
        pltpu.async_copy(xflat.at[iv.at[2]], g2, sem)
        pltpu.async_copy(xflat.at[iv.at[3]], g3, sem)

    def wait_gathers(slot):
        iv, wv, g0, g1, g2, g3, sem = slot
        pltpu.make_async_copy(xflat.at[iv.at[0]], g0, sem).wait()
        pltpu.make_async_copy(xflat.at[iv.at[1]], g1, sem).wait()
        pltpu.make_async_copy(xflat.at[iv.at[2]], g2, sem).wait()
        pltpu.make_async_copy(xflat.at[iv.at[3]], g3, sem).wait()

    def compute(slot):
        iv, wv, g0, g1, g2, g3, sem = slot

        def grp(g, _):
            p0 = g * 16
            wa_v = wv[0, pl.ds(p0, 16)]
            wb_v = wv[1, pl.ds(p0, 16)]
            wc_v = wv[2, pl.ds(p0, 16)]
            wd_v = wv[3, pl.ds(p0, 16)]
            for j in range(16):
                p = p0 + j
                wa, wb, wc, wd = wa_v[j], wb_v[j], wc_v[j], wd_v[j]
                for cc in range(CV):
                    sl = pl.ds(cc * 16, 16)
                    ob[p, sl] = (g0[p, sl] * wa + g1[p, sl] * wb
                                 + g2[p, sl] * wc + g3[p, sl] * wd)
            return 0

        lax.fori_loop(0, CH // 16, grp, 0)

    def wait_out(ci):
        base = tile_base + ci * CH
        pltpu.make_async_copy(ob, out_hbm.at[pl.ds(base, CH)], osem).wait()

    def step(ci, slot):
        wait_gathers(slot)

        @pl.when(ci >= 1)
        def _():
            wait_out(ci - 1)

        compute(slot)
        pltpu.async_copy(ob, out_hbm.at[pl.ds(tile_base + ci * CH, CH)], osem)

        @pl.when(ci + 2 < N_CHUNKS)
        def _():
            load_and_fire(ci + 2, slot)

    load_and_fire(0, slots[0])
    load_and_fire(1, slots[1])

    def pair(it, _):
        step(it * 2, slots[0])
        step(it * 2 + 1, slots[1])
        return 0

    lax.fori_loop(0, N_CHUNKS // 2, pair, 0)
    wait_out(N_CHUNKS - 1)


@functools.cache
def _make_sample():
    mesh = plsc.VectorSubcoreMesh(
        core_axis_name="c", subcore_axis_name="s", num_cores=2, num_subcores=16)

    @functools.partial(
        pl.kernel,
        out_type=jax.ShapeDtypeStruct((NPIX, C), jnp.float32),
        mesh=mesh,
        compiler_params=pltpu.CompilerParams(use_tc_tiling_on_sc=False),
        scratch_types=[
            t
            for _ in range(2)
            for t in [
                pltpu.VMEM((4, CH), jnp.int32),
                pltpu.VMEM((4, CH), jnp.float32),
                pltpu.VMEM((CH, XP), jnp.float32),
                pltpu.VMEM((CH, XP), jnp.float32),
                pltpu.VMEM((CH, XP), jnp.float32),
                pltpu.VMEM((CH, XP), jnp.float32),
                pltpu.SemaphoreType.DMA,
            ]
        ] + [
            pltpu.VMEM((CH, C), jnp.float32),
            pltpu.SemaphoreType.DMA,
        ],
    )
    def _sample(xflat, idx_hbm, wt_hbm, out_hbm, *scratch):
        _sc_body(xflat, idx_hbm, wt_hbm, out_hbm, *scratch)

    return _sample


def kernel(X, W_loc, b_loc):
    pooled, xpad = _pooled(X)
    idx, wt = _grid_tables(pooled, W_loc, b_loc.reshape(1, 6))
    out = _make_sample()(xpad, idx, wt)
    return out.reshape(B, OUT_H, OUT_W, C)


# speedup vs baseline: 1.1965x; 1.1965x over previous
import functools

import jax
import jax.numpy as jnp
from jax import lax
from jax.experimental import pallas as pl
from jax.experimental.pallas import tpu as pltpu
from jax.experimental.pallas import tpu_sc as plsc

B, H, W, C = 4, 224, 224, 96
OUT_H, OUT_W = 224, 224
NPIX = B * OUT_H * OUT_W
NW = 32
PIX_PER_TILE = NPIX // NW
CH = 128
N_CHUNKS = PIX_PER_TILE // CH
TROW = 128
N_TROWS = NPIX // TROW
ROWS_PER_STEP = 32
CV = C // 16


def _pool_body(x_ref, out_ref):
    h = pl.program_id(0)
    s = jnp.sum(x_ref[...], axis=(1, 2)) * (1.0 / (H * W))

    @pl.when(h == 0)
    def _():
        out_ref[...] = s

    @pl.when(h != 0)
    def _():
        out_ref[...] = out_ref[...] + s


def _pooled(X):
    hc = 28
    return pl.pallas_call(
        _pool_body,
        grid=(H // hc,),
        in_specs=[pl.BlockSpec((B, hc, W, C), lambda h: (0, h, 0, 0))],
        out_specs=pl.BlockSpec((B, C), lambda h: (0, 0)),
        out_shape=jax.ShapeDtypeStruct((B, C), jnp.float32),
    )(X)


def _grid_body(pooled_ref, wloc_ref, bloc_ref, idx_ref, wt_ref):
    b = pl.program_id(0)
    def _bf(v):
        return v.astype(jnp.bfloat16).astype(jnp.float32)

    th_all = jnp.dot(pooled_ref[...].astype(jnp.bfloat16),
                     wloc_ref[...].astype(jnp.bfloat16),
                     preferred_element_type=jnp.float32) + bloc_ref[...]
    sel = lax.broadcasted_iota(jnp.int32, (B, 6), 0) == b
    th = jnp.sum(jnp.where(sel, th_all, 0.0), axis=0)
    t00, t01, t02 = _bf(th[0]), _bf(th[1]), _bf(th[2])
    t10, t11, t12 = _bf(th[3]), _bf(th[4]), _bf(th[5])

    shp = (NPIX // B // TROW, TROW)
    p_loc = (lax.broadcasted_iota(jnp.int32, shp, 0) * TROW
             + lax.broadcasted_iota(jnp.int32, shp, 1))
    p2 = p_loc >> 5
    row_i = (p2 * 9363) >> 16
    col_i = p_loc - row_i * OUT_W
    col = col_i.astype(jnp.float32)
    row = row_i.astype(jnp.float32)
    sc = col / jnp.float32(OUT_W - 1)
    sr = row / jnp.float32(OUT_H - 1)
    xg = _bf(sc - (1.0 - sc))
    yg = _bf(sr - (1.0 - sr))
    x_s = t00 * xg + t01 * yg + t02
    y_s = t10 * xg + t11 * yg + t12
    x = 0.5 * (x_s + 1.0) * jnp.float32(W)
    y = 0.5 * (y_s + 1.0) * jnp.float32(H)

    xf = jnp.floor(x)
    yf = jnp.floor(y)
    x0f = jnp.clip(xf, 0.0, W - 1)
    x1f = jnp.clip(xf + 1.0, 0.0, W - 1)
    y0f = jnp.clip(yf, 0.0, H - 1)
    y1f = jnp.clip(yf + 1.0, 0.0, H - 1)
    x0 = x0f.astype(jnp.int32)
    x1 = x1f.astype(jnp.int32)
    y0 = y0f.astype(jnp.int32)
    y1 = y1f.astype(jnp.int32)

    wa = (x1f - x) * (y1f - y)
    wb = (x1f - x) * (y - y0f)
    wc = (x - x0f) * (y1f - y)
    wd = (x - x0f) * (y - y0f)

    base = b * (H * W)
    ia = base + y0 * W + x0
    ib = base + y1 * W + x0
    ic = base + y0 * W + x1
    id_ = base + y1 * W + x1

    idx_ref[...] = jnp.stack([ia, ib, ic, id_], axis=0)
    wt_ref[...] = jnp.stack([wa, wb, wc, wd], axis=0)


def _grid_tables(pooled, W_loc, b_loc2):
    cpb = NPIX // B // TROW
    return pl.pallas_call(
        _grid_body,
        grid=(B,),
        in_specs=[
            pl.BlockSpec((B, C), lambda b: (0, 0)),
            pl.BlockSpec((C, 6), lambda b: (0, 0)),
            pl.BlockSpec((1, 6), lambda b: (0, 0)),
        ],
        out_specs=[
            pl.BlockSpec((4, cpb, TROW), lambda b: (0, b, 0)),
            pl.BlockSpec((4, cpb, TROW), lambda b: (0, b, 0)),
        ],
        out_shape=[
            jax.ShapeDtypeStruct((4, N_TROWS, TROW), jnp.int32),
            jax.ShapeDtypeStruct((4, N_TROWS, TROW), jnp.float32),
        ],
    )(pooled, W_loc, b_loc2)


def _sc_body(xflat, idx_hbm, wt_hbm, out_hbm, *s):
    slots = (s[:7], s[7:14])
    ob, osem = s[14], s[15]
    wid = lax.axis_index("s") * 2 + lax.axis_index("c")
    tile_base = wid * PIX_PER_TILE

    def load_and_fire(ci, slot):
        iv, wv, g0, g1, g2, g3, sem = slot
        g = wid * N_CHUNKS + ci
        pltpu.sync_copy(idx_hbm.at[:, g], iv)
        pltpu.sync_copy(wt_hbm.at[:, g], wv)
        pltpu.async_copy(xflat.at[iv.at[0]], g0, sem)
        pltpu.async_copy(xflat.at[iv.at[1]], g1, sem)
        pltpu.async_copy(xflat.at[iv.at[2]], g2, sem)
        pltpu.async_copy(xflat.at[iv.at[3]], g3, sem)

    def wait_gathers(slot):
        iv, wv, g0, g1, g2, g3, sem = slot
        pltpu.make_async_copy(xflat.at[iv.at[0]], g0, sem).wait()
        pltpu.make_async_copy(xflat.at[iv.at[1]], g1, sem).wait()
        pltpu.make_async_copy(xflat.at[iv.at[2]], g2, sem).wait()
        pltpu.make_async_copy(xflat.at[iv.at[3]], g3, sem).wait()

    def compute(slot):
        iv, wv, g0, g1, g2, g3, sem = slot

        def grp(g, _):
            p0 = g * 16
            wa_v = wv[0, pl.ds(p0, 16)]
            wb_v = wv[1, pl.ds(p0, 16)]
            wc_v = wv[2, pl.ds(p0, 16)]
            wd_v = wv[3, pl.ds(p0, 16)]
            for j in range(16):
                p = p0 + j
                wa, wb, wc, wd = wa_v[j], wb_v[j], wc_v[j], wd_v[j]
                for cc in range(CV):
                    sl = pl.ds(cc * 16, 16)
                    ob[p, sl] = (g0[p, sl] * wa + g1[p, sl] * wb
                                 + g2[p, sl] * wc + g3[p, sl] * wd)
            return 0

        lax.fori_loop(0, CH // 16, grp, 0)

    def wait_out(ci):
        base = tile_base + ci * CH
        pltpu.make_async_copy(ob, out_hbm.at[pl.ds(base, CH)], osem).wait()

    def step(ci, slot):
        wait_gathers(slot)

        @pl.when(ci >= 1)
        def _():
            wait_out(ci - 1)

        compute(slot)
        pltpu.async_copy(ob, out_hbm.at[pl.ds(tile_base + ci * CH, CH)], osem)

        @pl.when(ci + 2 < N_CHUNKS)
        def _():
            load_and_fire(ci + 2, slot)

    load_and_fire(0, slots[0])
    load_and_fire(1, slots[1])

    def pair(it, _):
        step(it * 2, slots[0])
        step(it * 2 + 1, slots[1])
        return 0

    lax.fori_loop(0, N_CHUNKS // 2, pair, 0)
    step(N_CHUNKS - 1, slots[0])
    wait_out(N_CHUNKS - 1)


@functools.cache
def _make_sample():
    mesh = plsc.VectorSubcoreMesh(
        core_axis_name="c", subcore_axis_name="s", num_cores=2, num_subcores=16)

    @functools.partial(
        pl.kernel,
        out_type=jax.ShapeDtypeStruct((NPIX, C), jnp.float32),
        mesh=mesh,
        compiler_params=pltpu.CompilerParams(use_tc_tiling_on_sc=False),
        scratch_types=[
            t
            for _ in range(2)
            for t in [
                pltpu.VMEM((4, CH), jnp.int32),
                pltpu.VMEM((4, CH), jnp.float32),
                pltpu.VMEM((CH, C), jnp.float32),
                pltpu.VMEM((CH, C), jnp.float32),
                pltpu.VMEM((CH, C), jnp.float32),
                pltpu.VMEM((CH, C), jnp.float32),
                pltpu.SemaphoreType.DMA,
            ]
        ] + [
            pltpu.VMEM((CH, C), jnp.float32),
            pltpu.SemaphoreType.DMA,
        ],
    )
    def _sample(xflat, idx_hbm, wt_hbm, out_hbm, *scratch):
        _sc_body(xflat, idx_hbm, wt_hbm, out_hbm, *scratch)

    return _sample


def kernel(X, W_loc, b_loc):
    pooled = _pooled(X)
    idx, wt = _grid_tables(pooled, W_loc, b_loc.reshape(1, 6))
    out = _make_sample()(X.reshape(NPIX, C), idx, wt)
    return out.reshape(B, OUT_H, OUT_W, C)


# speedup vs baseline: 1.2469x; 1.0421x over previous
import functools

import jax
import jax.numpy as jnp
from jax import lax
from jax.experimental import pallas as pl
from jax.experimental.pallas import tpu as pltpu
from jax.experimental.pallas import tpu_sc as plsc

B, H, W, C = 4, 224, 224, 96
OUT_H, OUT_W = 224, 224
NPIX = B * OUT_H * OUT_W
NW = 32
PIX_PER_TILE = NPIX // NW
CH = 128
N_CHUNKS = PIX_PER_TILE // CH
TROW = 128
N_TROWS = NPIX // TROW
ROWS_PER_STEP = 32
CV = C // 16


def _pool_body(x_ref, out_ref):
    h = pl.program_id(0)
    s = jnp.sum(x_ref[...], axis=(1, 2)) * (1.0 / (H * W))

    @pl.when(h == 0)
    def _():
        out_ref[...] = s

    @pl.when(h != 0)
    def _():
        out_ref[...] = out_ref[...] + s


def _pooled(X):
    hc = 28
    return pl.pallas_call(
        _pool_body,
        grid=(H // hc,),
        in_specs=[pl.BlockSpec((B, hc, W, C), lambda h: (0, h, 0, 0))],
        out_specs=pl.BlockSpec((B, C), lambda h: (0, 0)),
        out_shape=jax.ShapeDtypeStruct((B, C), jnp.float32),
    )(X)


def _grid_body(pooled_ref, wloc_ref, bloc_ref, tab_ref):
    b = pl.program_id(0)
    def _bf(v):
        return v.astype(jnp.bfloat16).astype(jnp.float32)

    th_all = jnp.dot(pooled_ref[...].astype(jnp.bfloat16),
                     wloc_ref[...].astype(jnp.bfloat16),
                     preferred_element_type=jnp.float32) + bloc_ref[...]
    sel = lax.broadcasted_iota(jnp.int32, (B, 6), 0) == b
    th = jnp.sum(jnp.where(sel, th_all, 0.0), axis=0)
    t00, t01, t02 = _bf(th[0]), _bf(th[1]), _bf(th[2])
    t10, t11, t12 = _bf(th[3]), _bf(th[4]), _bf(th[5])

    shp = (NPIX // B // TROW, TROW)
    p_loc = (lax.broadcasted_iota(jnp.int32, shp, 0) * TROW
             + lax.broadcasted_iota(jnp.int32, shp, 1))
    p2 = p_loc >> 5
    row_i = (p2 * 9363) >> 16
    col_i = p_loc - row_i * OUT_W
    col = col_i.astype(jnp.float32)
    row = row_i.astype(jnp.float32)
    sc = col / jnp.float32(OUT_W - 1)
    sr = row / jnp.float32(OUT_H - 1)
    xg = _bf(sc - (1.0 - sc))
    yg = _bf(sr - (1.0 - sr))
    x_s = t00 * xg + t01 * yg + t02
    y_s = t10 * xg + t11 * yg + t12
    x = 0.5 * (x_s + 1.0) * jnp.float32(W)
    y = 0.5 * (y_s + 1.0) * jnp.float32(H)

    xf = jnp.floor(x)
    yf = jnp.floor(y)
    x0f = jnp.clip(xf, 0.0, W - 1)
    x1f = jnp.clip(xf + 1.0, 0.0, W - 1)
    y0f = jnp.clip(yf, 0.0, H - 1)
    y1f = jnp.clip(yf + 1.0, 0.0, H - 1)
    x0 = x0f.astype(jnp.int32)
    x1 = x1f.astype(jnp.int32)
    y0 = y0f.astype(jnp.int32)
    y1 = y1f.astype(jnp.int32)

    wa = (x1f - x) * (y1f - y)
    wb = (x1f - x) * (y - y0f)
    wc = (x - x0f) * (y1f - y)
    wd = (x - x0f) * (y - y0f)

    base = b * (H * W)
    ia = base + y0 * W + x0
    ib = base + y1 * W + x0
    ic = base + y0 * W + x1
    id_ = base + y1 * W + x1

    wbits = lax.bitcast_convert_type(jnp.stack([wa, wb, wc, wd], axis=0),
                                     jnp.int32)
    tab_ref[...] = jnp.concatenate(
        [jnp.stack([ia, ib, ic, id_], axis=0), wbits], axis=0)


def _grid_tables(pooled, W_loc, b_loc2):
    cpb = NPIX // B // TROW
    return pl.pallas_call(
        _grid_body,
        grid=(B,),
        in_specs=[
            pl.BlockSpec((B, C), lambda b: (0, 0)),
            pl.BlockSpec((C, 6), lambda b: (0, 0)),
            pl.BlockSpec((1, 6), lambda b: (0, 0)),
        ],
        out_specs=pl.BlockSpec((8, cpb, TROW), lambda b: (0, b, 0)),
        out_shape=jax.ShapeDtypeStruct((8, N_TROWS, TROW), jnp.int32),
    )(pooled, W_loc, b_loc2)


def _sc_body(xflat, tab_hbm, out_hbm, *s):
    slots = (s[:6], s[6:12])
    ob, osem = s[12], s[13]
    wid = lax.axis_index("s") * 2 + lax.axis_index("c")
    tile_base = wid * PIX_PER_TILE

    def load_and_fire(ci, slot):
        iv, g0, g1, g2, g3, sem = slot
        g = wid * N_CHUNKS + ci
        pltpu.sync_copy(tab_hbm.at[:, g], iv)
        pltpu.async_copy(xflat.at[iv.at[0]], g0, sem)
        pltpu.async_copy(xflat.at[iv.at[1]], g1, sem)
        pltpu.async_copy(xflat.at[iv.at[2]], g2, sem)
        pltpu.async_copy(xflat.at[iv.at[3]], g3, sem)

    def wait_gathers(slot):
        iv, g0, g1, g2, g3, sem = slot
        pltpu.make_async_copy(xflat.at[iv.at[0]], g0, sem).wait()
        pltpu.make_async_copy(xflat.at[iv.at[1]], g1, sem).wait()
        pltpu.make_async_copy(xflat.at[iv.at[2]], g2, sem).wait()
        pltpu.make_async_copy(xflat.at[iv.at[3]], g3, sem).wait()

    def compute(slot):
        iv, g0, g1, g2, g3, sem = slot

        def grp(g, _):
            p0 = g * 16

            def wrow(t):
                return lax.bitcast_convert_type(iv[4 + t, pl.ds(p0, 16)],
                                                jnp.float32)

            wa_v, wb_v, wc_v, wd_v = wrow(0), wrow(1), wrow(2), wrow(3)
            for j in range(16):
                p = p0 + j
                wa, wb, wc, wd = wa_v[j], wb_v[j], wc_v[j], wd_v[j]
                for cc in range(CV):
                    sl = pl.ds(cc * 16, 16)
                    ob[p, sl] = (g0[p, sl] * wa + g1[p, sl] * wb
                                 + g2[p, sl] * wc + g3[p, sl] * wd)
            return 0

        lax.fori_loop(0, CH // 16, grp, 0)

    def wait_out(ci):
        base = tile_base + ci * CH
        pltpu.make_async_copy(ob, out_hbm.at[pl.ds(base, CH)], osem).wait()

    def step(ci, slot):
        wait_gathers(slot)

        @pl.when(ci >= 1)
        def _():
            wait_out(ci - 1)

        compute(slot)
        pltpu.async_copy(ob, out_hbm.at[pl.ds(tile_base + ci * CH, CH)], osem)

        @pl.when(ci + 2 < N_CHUNKS)
        def _():
            load_and_fire(ci + 2, slot)

    load_and_fire(0, slots[0])
    load_and_fire(1, slots[1])

    def pair(it, _):
        step(it * 2, slots[0])
        step(it * 2 + 1, slots[1])
        return 0

    lax.fori_loop(0, N_CHUNKS // 2, pair, 0)
    step(N_CHUNKS - 1, slots[0])
    wait_out(N_CHUNKS - 1)


@functools.cache
def _make_sample():
    mesh = plsc.VectorSubcoreMesh(
        core_axis_name="c", subcore_axis_name="s", num_cores=2, num_subcores=16)

    @functools.partial(
        pl.kernel,
        out_type=jax.ShapeDtypeStruct((NPIX, C), jnp.float32),
        mesh=mesh,
        compiler_params=pltpu.CompilerParams(use_tc_tiling_on_sc=False),
        scratch_types=[
            t
            for _ in range(2)
            for t in [
                pltpu.VMEM((8, CH), jnp.int32),
                pltpu.VMEM((CH, C), jnp.float32),
                pltpu.VMEM((CH, C), jnp.float32),
                pltpu.VMEM((CH, C), jnp.float32),
                pltpu.VMEM((CH, C), jnp.float32),
                pltpu.SemaphoreType.DMA,
            ]
        ] + [
            pltpu.VMEM((CH, C), jnp.float32),
            pltpu.SemaphoreType.DMA,
        ],
    )
    def _sample(xflat, tab_hbm, out_hbm, *scratch):
        _sc_body(xflat, tab_hbm, out_hbm, *scratch)

    return _sample


def kernel(X, W_loc, b_loc):
    pooled = _pooled(X)
    tab = _grid_tables(pooled, W_loc, b_loc.reshape(1, 6))
    out = _make_sample()(X.reshape(NPIX, C), tab)
    return out.reshape(B, OUT_H, OUT_W, C)
